# Initial kernel scaffold; baseline (speedup 1.0000x reference)
import jax
import jax.numpy as jnp
from jax.experimental import pallas as pl


def kernel(X, edge_index, edge_weight, W1, b1, Wmu, bmu, Wls, bls):
    raise NotImplementedError("write your pallas kernel here")


# speedup vs baseline: 16.2318x; 16.2318x over previous
import functools

import jax
import jax.numpy as jnp
from jax import lax
from jax.experimental import pallas as pl
from jax.experimental.pallas import tpu as pltpu
from jax.experimental.pallas import tpu_sc as plsc

N = 10000
E = 320000
C = 128
OC = 64
NP = 10240
NC = 2
NS = 16
NW = NC * NS
EPW = E // NW
K = 80
NCHUNK = EPW // K
RPT = NP // NS

_f32 = jnp.float32
_i32 = jnp.int32

_MESH = dict(
    mesh=plsc.VectorSubcoreMesh(core_axis_name="c", subcore_axis_name="s",
                                num_cores=NC, num_subcores=NS),
    compiler_params=pltpu.CompilerParams(needs_layout_passes=False),
)


def _wid_tile():
    cid = lax.axis_index("c")
    sid = lax.axis_index("s")
    return sid * NC + cid, sid, cid


@functools.partial(
    pl.kernel,
    out_type=jax.ShapeDtypeStruct((NC, 2, NP), _f32),
    scratch_types=[
        pltpu.VMEM((K,), _i32),
        pltpu.VMEM((K,), _f32),
        pltpu.VMEM((K,), _f32),
        pltpu.VMEM_SHARED((NP,), _f32),
        pltpu.VMEM_SHARED((NP,), _f32),
    ],
    **_MESH,
)
def _deg_kernel(dst_hbm, ew_hbm, zeros_hbm, deg_out, dstv, ewv, onesv,
                degw_sp, deg1_sp):
    wid, tile, cid = _wid_tile()
    sl = pl.ds(tile * RPT, RPT)
    pltpu.sync_copy(zeros_hbm.at[sl], degw_sp.at[sl])
    pltpu.sync_copy(zeros_hbm.at[sl], deg1_sp.at[sl])
    ones16 = jnp.ones((16,), _f32)
    for j in range(K // 16):
        onesv[pl.ds(j * 16, 16)] = ones16
    plsc.subcore_barrier()

    def chunk(i, carry):
        base = wid * EPW + i * K
        pltpu.sync_copy(dst_hbm.at[pl.ds(base, K)], dstv)
        pltpu.sync_copy(ew_hbm.at[pl.ds(base, K)], ewv)
        pltpu.sync_copy(ewv, degw_sp.at[dstv], add=True)
        pltpu.sync_copy(onesv, deg1_sp.at[dstv], add=True)
        return carry

    lax.fori_loop(0, NCHUNK, chunk, 0)
    plsc.subcore_barrier()
    pltpu.sync_copy(degw_sp.at[sl], deg_out.at[cid, 0, sl])
    pltpu.sync_copy(deg1_sp.at[sl], deg_out.at[cid, 1, sl])


def _make_agg(scaled: bool):
    scratch = [
        pltpu.VMEM((K,), _i32),
        pltpu.VMEM((K,), _i32),
        pltpu.VMEM((K, C), _f32),
        pltpu.SemaphoreType.DMA,
        pltpu.VMEM_SHARED((NP, C), _f32),
    ]
    if scaled:
        scratch += [
            pltpu.VMEM((K,), _f32),
            pltpu.VMEM((NP,), _f32),
        ]

    def body(src_hbm, dst_hbm, *rest):
        if scaled:
            (ew_hbm, dinv_hbm, x_hbm, zeros_hbm, z_out,
             srcv, dstv, rows, sem, accum, ewv, dinvt) = rest
        else:
            (x_hbm, zeros_hbm, z_out,
             srcv, dstv, rows, sem, accum) = rest
        wid, tile, cid = _wid_tile()
        pltpu.sync_copy(zeros_hbm.at[pl.ds(tile * RPT, RPT)],
                        accum.at[pl.ds(tile * RPT, RPT)])
        if scaled:
            pltpu.sync_copy(dinv_hbm, dinvt)
        plsc.subcore_barrier()

        def chunk(i, carry):
            base = wid * EPW + i * K
            pltpu.sync_copy(src_hbm.at[pl.ds(base, K)], srcv)
            pltpu.sync_copy(dst_hbm.at[pl.ds(base, K)], dstv)
            if scaled:
                pltpu.sync_copy(ew_hbm.at[pl.ds(base, K)], ewv)
            pltpu.async_copy(x_hbm.at[srcv], rows, sem).wait()
            if scaled:
                for j in range(K // 16):
                    s16 = srcv[pl.ds(j * 16, 16)]
                    d16 = dstv[pl.ds(j * 16, 16)]
                    e16 = ewv[pl.ds(j * 16, 16)]
                    n16 = (plsc.load_gather(dinvt, [s16]) * e16 *
                           plsc.load_gather(dinvt, [d16]))
                    for l in range(16):
                        e = j * 16 + l
                        s = n16[l]
                        for cc in range(C // 16):
                            sl = pl.ds(cc * 16, 16)
                            rows[e, sl] = rows[e, sl] * s
            pltpu.sync_copy(rows, accum.at[dstv], add=True)
            return carry

        lax.fori_loop(0, NCHUNK, chunk, 0)
        plsc.subcore_barrier()
        pltpu.sync_copy(accum.at[pl.ds(tile * RPT, RPT)],
                        z_out.at[cid, pl.ds(tile * RPT, RPT)])

    return pl.kernel(
        body,
        out_type=jax.ShapeDtypeStruct((NC, NP, C), _f32),
        scratch_types=scratch,
        **_MESH,
    )


_agg_scaled = _make_agg(True)
_agg_plain = _make_agg(False)


def _dinv_body(deg_ref, dinvw_ref, dinv1_ref):
    d = deg_ref[0] + deg_ref[1]
    dinvw_ref[...] = lax.rsqrt(d[0] + 1.0)
    dinv1_ref[...] = lax.rsqrt(d[1] + 1.0)


_dinv_call = pl.pallas_call(
    _dinv_body,
    out_shape=[
        jax.ShapeDtypeStruct((NP,), _f32),
        jax.ShapeDtypeStruct((NP,), _f32),
    ],
)

_RB = 2000


def _h_body(z_ref, x_ref, dw_ref, d1_ref, w1_ref, b1_ref, h_ref, y2_ref):
    dw = dw_ref[...]
    s1 = z_ref[0] + z_ref[1] + dw * dw * x_ref[...]
    h = jnp.dot(s1, w1_ref[...], preferred_element_type=_f32) + b1_ref[...]
    h = jnp.maximum(h, 0.0)
    h_ref[...] = h
    y2_ref[...] = d1_ref[...] * h


_h_call = pl.pallas_call(
    _h_body,
    grid=(N // _RB,),
    in_specs=[
        pl.BlockSpec((NC, _RB, C), lambda i: (0, i, 0)),
        pl.BlockSpec((_RB, C), lambda i: (i, 0)),
        pl.BlockSpec((_RB, 1), lambda i: (i, 0)),
        pl.BlockSpec((_RB, 1), lambda i: (i, 0)),
        pl.BlockSpec((C, C), lambda i: (0, 0)),
        pl.BlockSpec((1, C), lambda i: (0, 0)),
    ],
    out_specs=[
        pl.BlockSpec((_RB, C), lambda i: (i, 0)),
        pl.BlockSpec((_RB, C), lambda i: (i, 0)),
    ],
    out_shape=[
        jax.ShapeDtypeStruct((N, C), _f32),
        jax.ShapeDtypeStruct((N, C), _f32),
    ],
)


def _out_body(z_ref, h_ref, d1_ref, wmu_ref, bmu_ref, wls_ref, bls_ref,
              mu_ref, ls_ref):
    d1 = d1_ref[...]
    g = d1 * (z_ref[0] + z_ref[1]) + d1 * d1 * h_ref[...]
    mu_ref[...] = jnp.dot(g, wmu_ref[...], preferred_element_type=_f32) + bmu_ref[...]
    ls_ref[...] = jnp.dot(g, wls_ref[...], preferred_element_type=_f32) + bls_ref[...]


_out_call = pl.pallas_call(
    _out_body,
    grid=(N // _RB,),
    in_specs=[
        pl.BlockSpec((NC, _RB, C), lambda i: (0, i, 0)),
        pl.BlockSpec((_RB, C), lambda i: (i, 0)),
        pl.BlockSpec((_RB, 1), lambda i: (i, 0)),
        pl.BlockSpec((C, OC), lambda i: (0, 0)),
        pl.BlockSpec((1, OC), lambda i: (0, 0)),
        pl.BlockSpec((C, OC), lambda i: (0, 0)),
        pl.BlockSpec((1, OC), lambda i: (0, 0)),
    ],
    out_specs=[
        pl.BlockSpec((_RB, OC), lambda i: (i, 0)),
        pl.BlockSpec((_RB, OC), lambda i: (i, 0)),
    ],
    out_shape=[
        jax.ShapeDtypeStruct((N, OC), _f32),
        jax.ShapeDtypeStruct((N, OC), _f32),
    ],
)


def kernel(X, edge_index, edge_weight, W1, b1, Wmu, bmu, Wls, bls):
    src = edge_index[0].astype(_i32)
    dst = edge_index[1].astype(_i32)
    ew = edge_weight.astype(_f32)
    zeros1 = jnp.zeros((NP,), _f32)
    zeros128 = jnp.zeros((NP, C), _f32)

    deg = _deg_kernel(dst, ew, zeros1)
    dinvw, dinv1 = _dinv_call(deg)
    dinvw_col = dinvw.reshape(NP, 1)
    dinv1_col = dinv1.reshape(NP, 1)
    z1 = _agg_scaled(src, dst, ew, dinvw, X, zeros128)
    h, y2 = _h_call(z1, X, dinvw_col, dinv1_col, W1, b1.reshape(1, C))
    z2 = _agg_plain(src, dst, y2, zeros128)
    mu, ls = _out_call(z2, h, dinv1_col, Wmu, bmu.reshape(1, OC),
                       Wls, bls.reshape(1, OC))
    return (mu, ls)


# speedup vs baseline: 28.3636x; 1.7474x over previous
import functools

import jax
import jax.numpy as jnp
from jax import lax
from jax.experimental import pallas as pl
from jax.experimental.pallas import tpu as pltpu
from jax.experimental.pallas import tpu_sc as plsc

N = 10000
E = 320000
C = 128
OC = 64
NP = 10240
NC = 2
NS = 16
NW = NC * NS
EPW = E // NW
K = 80
NCHUNK = EPW // K
RPT = NP // NS

_f32 = jnp.float32
_i32 = jnp.int32

_MESH = dict(
    mesh=plsc.VectorSubcoreMesh(core_axis_name="c", subcore_axis_name="s",
                                num_cores=NC, num_subcores=NS),
    compiler_params=pltpu.CompilerParams(needs_layout_passes=False),
)


def _wid_tile():
    cid = lax.axis_index("c")
    sid = lax.axis_index("s")
    return sid * NC + cid, sid, cid


@functools.partial(
    pl.kernel,
    out_type=jax.ShapeDtypeStruct((NC, 2, NP), _f32),
    scratch_types=[
        pltpu.VMEM((K,), _i32),
        pltpu.VMEM((K,), _i32),
        pltpu.VMEM((K,), _f32),
        pltpu.VMEM((K,), _f32),
        pltpu.VMEM((K,), _f32),
        pltpu.SemaphoreType.DMA,
        pltpu.SemaphoreType.DMA,
        pltpu.SemaphoreType.DMA,
        pltpu.SemaphoreType.DMA,
        pltpu.VMEM_SHARED((NP,), _f32),
        pltpu.VMEM_SHARED((NP,), _f32),
    ],
    **_MESH,
)
def _deg_kernel(dst_hbm, ew_hbm, zeros_hbm, deg_out,
                dstv0, dstv1, ewv0, ewv1, onesv,
                sem_i0, sem_i1, sem_s0, sem_s1,
                degw_sp, deg1_sp):
    wid, tile, cid = _wid_tile()
    dstv = (dstv0, dstv1)
    ewv = (ewv0, ewv1)
    sem_i = (sem_i0, sem_i1)
    sem_s = (sem_s0, sem_s1)
    sl = pl.ds(tile * RPT, RPT)
    pltpu.sync_copy(zeros_hbm.at[sl], degw_sp.at[sl])
    pltpu.sync_copy(zeros_hbm.at[sl], deg1_sp.at[sl])
    ones16 = jnp.ones((16,), _f32)
    for j in range(K // 16):
        onesv[pl.ds(j * 16, 16)] = ones16
    plsc.subcore_barrier()

    def issue_idx(b, c):
        base = wid * EPW + c * K
        pltpu.async_copy(dst_hbm.at[pl.ds(base, K)], dstv[b], sem_i[b])
        pltpu.async_copy(ew_hbm.at[pl.ds(base, K)], ewv[b], sem_i[b])

    def wait_idx(b):
        pltpu.make_async_copy(dst_hbm.at[pl.ds(0, K)], dstv[b], sem_i[b]).wait()
        pltpu.make_async_copy(ew_hbm.at[pl.ds(0, K)], ewv[b], sem_i[b]).wait()

    def issue_scatter(b):
        pltpu.async_copy(ewv[b], degw_sp.at[dstv[b]], sem_s[b], add=True)
        pltpu.async_copy(onesv, deg1_sp.at[dstv[b]], sem_s[b], add=True)

    def wait_scatter(b):
        pltpu.make_async_copy(ewv[b], degw_sp.at[dstv[b]], sem_s[b]).wait()
        pltpu.make_async_copy(onesv, deg1_sp.at[dstv[b]], sem_s[b]).wait()

    def handle(c, b, first=False, last=False):
        nb = 1 - b
        wait_idx(b)
        issue_scatter(b)
        if not last:
            if not first:
                wait_scatter(nb)
            issue_idx(nb, c + 1)

    issue_idx(0, 0)
    handle(0, 0, first=True)
    handle(1, 1)

    def pair(i2, carry):
        handle(2 * i2, 0)
        handle(2 * i2 + 1, 1)
        return carry

    lax.fori_loop(1, NCHUNK // 2, pair, 0)
    handle(NCHUNK - 1, 0, last=True)
    wait_scatter(1)
    wait_scatter(0)
    plsc.subcore_barrier()
    pltpu.sync_copy(degw_sp.at[sl], deg_out.at[cid, 0, sl])
    pltpu.sync_copy(deg1_sp.at[sl], deg_out.at[cid, 1, sl])


def _make_agg(scaled: bool):
    scratch = [
        pltpu.VMEM((K,), _i32),
        pltpu.VMEM((K,), _i32),
        pltpu.VMEM((K,), _i32),
        pltpu.VMEM((K,), _i32),
        pltpu.VMEM((K, C), _f32),
        pltpu.VMEM((K, C), _f32),
        pltpu.SemaphoreType.DMA,
        pltpu.SemaphoreType.DMA,
        pltpu.SemaphoreType.DMA,
        pltpu.SemaphoreType.DMA,
        pltpu.SemaphoreType.DMA,
        pltpu.SemaphoreType.DMA,
        pltpu.VMEM_SHARED((NP, C), _f32),
    ]
    if scaled:
        scratch += [
            pltpu.VMEM((K,), _f32),
            pltpu.VMEM((K,), _f32),
            pltpu.VMEM((NP,), _f32),
        ]

    def body(src_hbm, dst_hbm, *rest):
        if scaled:
            (ew_hbm, dinv_hbm, x_hbm, zeros_hbm, z_out,
             srcv0, srcv1, dstv0, dstv1, rows0, rows1,
             sem_i0, sem_i1, sem_g0, sem_g1, sem_s0, sem_s1,
             accum, ewv0, ewv1, dinvt) = rest
            ewv = (ewv0, ewv1)
        else:
            (x_hbm, zeros_hbm, z_out,
             srcv0, srcv1, dstv0, dstv1, rows0, rows1,
             sem_i0, sem_i1, sem_g0, sem_g1, sem_s0, sem_s1,
             accum) = rest
        srcv = (srcv0, srcv1)
        dstv = (dstv0, dstv1)
        rows = (rows0, rows1)
        sem_i = (sem_i0, sem_i1)
        sem_g = (sem_g0, sem_g1)
        sem_s = (sem_s0, sem_s1)
        wid, tile, cid = _wid_tile()
        pltpu.sync_copy(zeros_hbm.at[pl.ds(tile * RPT, RPT)],
                        accum.at[pl.ds(tile * RPT, RPT)])
        if scaled:
            pltpu.sync_copy(dinv_hbm, dinvt)
        plsc.subcore_barrier()

        def issue_idx(b, c):
            base = wid * EPW + c * K
            pltpu.async_copy(src_hbm.at[pl.ds(base, K)], srcv[b], sem_i[b])
            pltpu.async_copy(dst_hbm.at[pl.ds(base, K)], dstv[b], sem_i[b])
            if scaled:
                pltpu.async_copy(ew_hbm.at[pl.ds(base, K)], ewv[b], sem_i[b])

        def wait_idx(b):
            pltpu.make_async_copy(src_hbm.at[pl.ds(0, K)], srcv[b],
                                  sem_i[b]).wait()
            pltpu.make_async_copy(dst_hbm.at[pl.ds(0, K)], dstv[b],
                                  sem_i[b]).wait()
            if scaled:
                pltpu.make_async_copy(ew_hbm.at[pl.ds(0, K)], ewv[b],
                                      sem_i[b]).wait()

        def issue_gather(b):
            pltpu.async_copy(x_hbm.at[srcv[b]], rows[b], sem_g[b])

        def wait_gather(b):
            pltpu.make_async_copy(x_hbm.at[srcv[b]], rows[b], sem_g[b]).wait()

        def issue_scatter(b):
            pltpu.async_copy(rows[b], accum.at[dstv[b]], sem_s[b], add=True)

        def wait_scatter(b):
            pltpu.make_async_copy(rows[b], accum.at[dstv[b]], sem_s[b]).wait()

        def scale(b):
            for j in range(K // 16):
                s16 = srcv[b][pl.ds(j * 16, 16)]
                d16 = dstv[b][pl.ds(j * 16, 16)]
                e16 = ewv[b][pl.ds(j * 16, 16)]
                n16 = (plsc.load_gather(dinvt, [s16]) * e16 *
                       plsc.load_gather(dinvt, [d16]))
                for l in range(16):
                    e = j * 16 + l
                    s = n16[l]
                    for cc in range(C // 16):
                        sl = pl.ds(cc * 16, 16)
                        rows[b][e, sl] = rows[b][e, sl] * s

        def handle(c, b, first=False, last=False):
            nb = 1 - b
            if not last:
                if not first:
                    wait_scatter(nb)
                issue_idx(nb, c + 1)
                wait_idx(nb)
                issue_gather(nb)
            wait_gather(b)
            if scaled:
                scale(b)
            issue_scatter(b)

        issue_idx(0, 0)
        wait_idx(0)
        issue_gather(0)
        handle(0, 0, first=True)
        handle(1, 1)

        def pair(i2, carry):
            handle(2 * i2, 0)
            handle(2 * i2 + 1, 1)
            return carry

        lax.fori_loop(1, NCHUNK // 2, pair, 0)
        handle(NCHUNK - 1, 0, last=True)
        wait_scatter(1)
        wait_scatter(0)
        plsc.subcore_barrier()
        pltpu.sync_copy(accum.at[pl.ds(tile * RPT, RPT)],
                        z_out.at[cid, pl.ds(tile * RPT, RPT)])

    return pl.kernel(
        body,
        out_type=jax.ShapeDtypeStruct((NC, NP, C), _f32),
        scratch_types=scratch,
        **_MESH,
    )


_agg_scaled = _make_agg(True)
_agg_plain = _make_agg(False)


def _dinv_body(deg_ref, dinvw_ref, dinv1_ref):
    d = deg_ref[0] + deg_ref[1]
    dinvw_ref[...] = lax.rsqrt(d[0] + 1.0)
    dinv1_ref[...] = lax.rsqrt(d[1] + 1.0)


_dinv_call = pl.pallas_call(
    _dinv_body,
    out_shape=[
        jax.ShapeDtypeStruct((NP,), _f32),
        jax.ShapeDtypeStruct((NP,), _f32),
    ],
)

_RB = 2000


def _h_body(z_ref, x_ref, dw_ref, d1_ref, w1_ref, b1_ref, h_ref, y2_ref):
    dw = dw_ref[...]
    s1 = z_ref[0] + z_ref[1] + dw * dw * x_ref[...]
    h = jnp.dot(s1, w1_ref[...], preferred_element_type=_f32) + b1_ref[...]
    h = jnp.maximum(h, 0.0)
    h_ref[...] = h
    y2_ref[...] = d1_ref[...] * h


_h_call = pl.pallas_call(
    _h_body,
    grid=(N // _RB,),
    in_specs=[
        pl.BlockSpec((NC, _RB, C), lambda i: (0, i, 0)),
        pl.BlockSpec((_RB, C), lambda i: (i, 0)),
        pl.BlockSpec((_RB, 1), lambda i: (i, 0)),
        pl.BlockSpec((_RB, 1), lambda i: (i, 0)),
        pl.BlockSpec((C, C), lambda i: (0, 0)),
        pl.BlockSpec((1, C), lambda i: (0, 0)),
    ],
    out_specs=[
        pl.BlockSpec((_RB, C), lambda i: (i, 0)),
        pl.BlockSpec((_RB, C), lambda i: (i, 0)),
    ],
    out_shape=[
        jax.ShapeDtypeStruct((N, C), _f32),
        jax.ShapeDtypeStruct((N, C), _f32),
    ],
)


def _out_body(z_ref, h_ref, d1_ref, wmu_ref, bmu_ref, wls_ref, bls_ref,
              mu_ref, ls_ref):
    d1 = d1_ref[...]
    g = d1 * (z_ref[0] + z_ref[1]) + d1 * d1 * h_ref[...]
    mu_ref[...] = jnp.dot(g, wmu_ref[...], preferred_element_type=_f32) + bmu_ref[...]
    ls_ref[...] = jnp.dot(g, wls_ref[...], preferred_element_type=_f32) + bls_ref[...]


_out_call = pl.pallas_call(
    _out_body,
    grid=(N // _RB,),
    in_specs=[
        pl.BlockSpec((NC, _RB, C), lambda i: (0, i, 0)),
        pl.BlockSpec((_RB, C), lambda i: (i, 0)),
        pl.BlockSpec((_RB, 1), lambda i: (i, 0)),
        pl.BlockSpec((C, OC), lambda i: (0, 0)),
        pl.BlockSpec((1, OC), lambda i: (0, 0)),
        pl.BlockSpec((C, OC), lambda i: (0, 0)),
        pl.BlockSpec((1, OC), lambda i: (0, 0)),
    ],
    out_specs=[
        pl.BlockSpec((_RB, OC), lambda i: (i, 0)),
        pl.BlockSpec((_RB, OC), lambda i: (i, 0)),
    ],
    out_shape=[
        jax.ShapeDtypeStruct((N, OC), _f32),
        jax.ShapeDtypeStruct((N, OC), _f32),
    ],
)


def kernel(X, edge_index, edge_weight, W1, b1, Wmu, bmu, Wls, bls):
    src = edge_index[0].astype(_i32)
    dst = edge_index[1].astype(_i32)
    ew = edge_weight.astype(_f32)
    zeros1 = jnp.zeros((NP,), _f32)
    zeros128 = jnp.zeros((NP, C), _f32)

    deg = _deg_kernel(dst, ew, zeros1)
    dinvw, dinv1 = _dinv_call(deg)
    dinvw_col = dinvw.reshape(NP, 1)
    dinv1_col = dinv1.reshape(NP, 1)
    z1 = _agg_scaled(src, dst, ew, dinvw, X, zeros128)
    h, y2 = _h_call(z1, X, dinvw_col, dinv1_col, W1, b1.reshape(1, C))
    z2 = _agg_plain(src, dst, y2, zeros128)
    mu, ls = _out_call(z2, h, dinv1_col, Wmu, bmu.reshape(1, OC),
                       Wls, bls.reshape(1, OC))
    return (mu, ls)
